# baseline (speedup 1.0000x reference)
import functools
import jax
import jax.numpy as jnp
from jax import lax
from jax.experimental import pallas as pl
from jax.experimental.pallas import tpu as pltpu
from jax.experimental.pallas import tpu_sc as plsc

_N = 10000
_E = 320000
_FIN, _DIM, _C = 128, 16, 64
_NC, _NS, _L = 2, 16, 16
_NW = _NC * _NS
_SUP = 512
_TOTS = _E // _SUP
_SPW = _TOTS // _NW
_XTRA = _TOTS - _SPW * _NW
_SMAX = _SPW + 1
_NROWS = 10240
_RPT = _NROWS // _NS
_HR = 640
_HSL = _HR * _L // _NS


def _mesh():
    return plsc.VectorSubcoreMesh(
        core_axis_name="c", subcore_axis_name="s",
        num_cores=_NC, num_subcores=_NS)


def _worker_supers(wid):
    first_x = _NW - _XTRA
    base = _SPW * wid + lax.max(wid - first_x, 0)
    n = jnp.where(wid >= first_x, _SMAX, _SPW)
    return base, n


@functools.partial(
    pl.kernel,
    out_type=jax.ShapeDtypeStruct((_NC, _HR * _L, _L), jnp.float32),
    mesh=_mesh(),
    scratch_types=[
        pltpu.VMEM((_E // _NW // _L, _L), jnp.int32),
        pltpu.VMEM((_HR * _L,), jnp.float32),
        pltpu.VMEM((_NS, _HSL), jnp.float32),
        pltpu.VMEM((_HSL, _L), jnp.float32),
        pltpu.VMEM_SHARED((_NS, _HR * _L), jnp.float32),
    ],
    compiler_params=pltpu.CompilerParams(
        needs_layout_passes=False, use_tc_tiling_on_sc=False),
)
def _sc_deg(edge16_hbm, out_hbm, idx_v, hist_v, tmp_v, rep_v, stage_sh):
    cid = lax.axis_index("c")
    sid = lax.axis_index("s")
    wid = sid * _NC + cid
    epw = _E // _NW
    zero16 = jnp.zeros((_L,), jnp.float32)

    def zbody(i, _):
        hist_v[pl.ds(i * _L, _L)] = zero16
        return 0
    lax.fori_loop(0, _HR, zbody, 0)

    pltpu.sync_copy(edge16_hbm.at[1, pl.ds(wid * (epw // _L), epw // _L)],
                    idx_v)
    ones16 = jnp.ones((_L,), jnp.float32)

    def body(i, _):
        ii = idx_v[i]
        plsc.addupdate_scatter(hist_v, [ii], ones16)
        return 0
    lax.fori_loop(0, epw // _L, body, 0)

    pltpu.sync_copy(hist_v, stage_sh.at[sid])
    plsc.subcore_barrier()
    pltpu.sync_copy(stage_sh.at[:, pl.ds(sid * _HSL, _HSL)], tmp_v)

    def rbody(k, _):
        acc = tmp_v[0, pl.ds(k * _L, _L)]
        for t in range(1, _NS):
            acc = acc + tmp_v[t, pl.ds(k * _L, _L)]
        tmp_v[0, pl.ds(k * _L, _L)] = acc
        return 0
    lax.fori_loop(0, _HSL // _L, rbody, 0)

    lane = lax.iota(jnp.int32, _L)

    def bbody(k, _):
        v = tmp_v[0, pl.ds(k * _L, _L)]
        row = k * _L + lane
        for j in range(_L):
            col = jnp.full((_L,), j, jnp.int32)
            plsc.store_scatter(rep_v, [row, col], v)
        return 0
    lax.fori_loop(0, _HSL // _L, bbody, 0)
    pltpu.sync_copy(rep_v, out_hbm.at[cid, pl.ds(sid * _HSL, _HSL)])


def _make_agg(F, nbuf):
    @functools.partial(
        pl.kernel,
        out_type=jax.ShapeDtypeStruct((_NC, _NROWS, F), jnp.float32),
        mesh=_mesh(),
        scratch_types=[
            pltpu.VMEM((_SMAX, _SUP), jnp.int32),
            pltpu.VMEM((_SMAX, _SUP), jnp.int32),
            pltpu.VMEM((nbuf, _SUP, F), jnp.float32),
            pltpu.VMEM_SHARED((_NROWS, F), jnp.float32),
            pltpu.SemaphoreType.DMA,
            pltpu.SemaphoreType.DMA,
        ],
        compiler_params=pltpu.CompilerParams(
            needs_layout_passes=False, use_tc_tiling_on_sc=False),
    )
    def agg(edge4_hbm, tab_hbm, out_hbm, srcv, dstv, rows, acc_sh,
            gsem, ssem):
        cid = lax.axis_index("c")
        sid = lax.axis_index("s")
        wid = sid * _NC + cid
        base, nsc = _worker_supers(wid)
        zero16 = jnp.zeros((_L,), jnp.float32)

        def zbody(i, _):
            for k in range(F // _L):
                rows[0, i, pl.ds(k * _L, _L)] = zero16
            return 0
        lax.fori_loop(0, 128, zbody, 0)

        def zcopy(j, _):
            pltpu.sync_copy(
                rows.at[0, pl.ds(0, 128)],
                acc_sh.at[pl.ds(sid * _RPT + j * 128, 128)])
            return 0
        lax.fori_loop(0, _RPT // 128, zcopy, 0)

        pltpu.sync_copy(edge4_hbm.at[0, pl.ds(base, _SMAX)], srcv)
        pltpu.sync_copy(edge4_hbm.at[1, pl.ds(base, _SMAX)], dstv)
        plsc.subcore_barrier()

        def g_idx(c):
            return srcv.at[c]

        def s_idx(c):
            return dstv.at[c]

        for b in range(nbuf):
            pltpu.async_copy(tab_hbm.at[g_idx(b)], rows.at[b], gsem)

        def body(c, _):
            b = lax.rem(c, nbuf)
            pltpu.make_async_copy(tab_hbm.at[g_idx(c)], rows.at[b],
                                  gsem).wait()
            pltpu.async_copy(rows.at[b], acc_sh.at[s_idx(c)], ssem,
                             add=True)

            @pl.when(c >= 1)
            def _refill():
                p = c - 1
                pb = lax.rem(p, nbuf)
                pltpu.make_async_copy(rows.at[pb], acc_sh.at[s_idx(p)],
                                      ssem).wait()

                @pl.when(p + nbuf < nsc)
                def _issue():
                    pltpu.async_copy(tab_hbm.at[g_idx(p + nbuf)],
                                     rows.at[pb], gsem)
            return 0
        lax.fori_loop(0, nsc, body, 0)

        pltpu.make_async_copy(rows.at[lax.rem(nsc - 1, nbuf)],
                              acc_sh.at[s_idx(nsc - 1)], ssem).wait()
        plsc.subcore_barrier()
        pltpu.sync_copy(acc_sh.at[pl.ds(sid * _RPT, _RPT)],
                        out_hbm.at[cid, pl.ds(sid * _RPT, _RPT)])
    return agg


_sc_agg16 = _make_agg(_DIM, 3)
_sc_agg64 = _make_agg(_C, 2)


def _tc_mm1(x_ref, w_ref, o_ref):
    o_ref[...] = jnp.dot(x_ref[...], w_ref[...],
                         preferred_element_type=jnp.float32)


def _tc_prep(deg_ref, h_ref, hs_ref, dinv_ref):
    dsum = deg_ref[0, pl.ds(0, _N), :] + deg_ref[1, pl.ds(0, _N), :] + 1.0
    dinv = lax.rsqrt(dsum)
    dinv_ref[...] = dinv
    hs_ref[...] = h_ref[...] * dinv


def _tc_mid(a_ref, hs_ref, dinv_ref, w_ref, b_ref, gs_ref):
    dinv = dinv_ref[...]
    a = a_ref[0, pl.ds(0, _N), :] + a_ref[1, pl.ds(0, _N), :]
    pre = (a + hs_ref[...]) * dinv + b_ref[...]
    h1 = jnp.maximum(pre, 0.0)
    g = jnp.dot(h1, w_ref[...], preferred_element_type=jnp.float32)
    gs_ref[...] = g * dinv[:, 0:1]


def _tc_final(a_ref, gs_ref, dinv_ref, b_ref, logp_ref, out_ref):
    dinv = dinv_ref[:, 0:1]
    a = a_ref[0, pl.ds(0, _N), :] + a_ref[1, pl.ds(0, _N), :]
    out = (a + gs_ref[...]) * dinv + b_ref[...]
    m = jnp.max(out, axis=1, keepdims=True)
    lse = jnp.log(jnp.sum(jnp.exp(out - m), axis=1, keepdims=True)) + m
    out_ref[...] = out
    logp_ref[...] = out - lse


@jax.jit
def _run(x, edge_index, W1, b1, W2, b2):
    edge4 = edge_index.reshape(2, _TOTS, _SUP)
    edge16 = edge_index.reshape(2, _E // _L, _L)

    degp = _sc_deg(edge16)
    h = pl.pallas_call(
        _tc_mm1,
        out_shape=jax.ShapeDtypeStruct((_N, _DIM), jnp.float32),
    )(x, W1)

    hs, dinv = pl.pallas_call(
        _tc_prep,
        out_shape=(jax.ShapeDtypeStruct((_N, _DIM), jnp.float32),
                   jax.ShapeDtypeStruct((_N, _DIM), jnp.float32)),
    )(degp, h)

    a1 = _sc_agg16(edge4, hs)
    gs = pl.pallas_call(
        _tc_mid,
        out_shape=jax.ShapeDtypeStruct((_N, _C), jnp.float32),
    )(a1, hs, dinv, W2, b1.reshape(1, _DIM))

    a2 = _sc_agg64(edge4, gs)
    logp, out = pl.pallas_call(
        _tc_final,
        out_shape=(jax.ShapeDtypeStruct((_N, _C), jnp.float32),
                   jax.ShapeDtypeStruct((_N, _C), jnp.float32)),
    )(a2, gs, dinv, b2.reshape(1, _C))
    return logp, out


def kernel(x, edge_index, W1, b1, W2, b2):
    return _run(x, edge_index, W1, b1, W2, b2)


# speedup vs baseline: 1.0675x; 1.0675x over previous
import functools
import jax
import jax.numpy as jnp
from jax import lax
from jax.experimental import pallas as pl
from jax.experimental.pallas import tpu as pltpu
from jax.experimental.pallas import tpu_sc as plsc

_N = 10000
_E = 320000
_FIN, _DIM, _C = 128, 16, 64
_NC, _NS, _L = 2, 16, 16
_NW = _NC * _NS
_CHUNK = 128
_TOTC = _E // _CHUNK
_CPW = _TOTC // _NW
_XTRA = _TOTC - _CPW * _NW
_CMAX = _CPW + 1
_NROWS = 10240
_RPT = _NROWS // _NS
_HR = 640
_HSL = _HR * _L // _NS


def _mesh():
    return plsc.VectorSubcoreMesh(
        core_axis_name="c", subcore_axis_name="s",
        num_cores=_NC, num_subcores=_NS)


def _worker_chunks(wid):
    first_x = _NW - _XTRA
    base = _CPW * wid + lax.max(wid - first_x, 0)
    n = jnp.where(wid >= first_x, _CMAX, _CPW)
    return base, n


@functools.partial(
    pl.kernel,
    out_type=jax.ShapeDtypeStruct((_NC, _HR * _L, _L), jnp.float32),
    mesh=_mesh(),
    scratch_types=[
        pltpu.VMEM((_E // _NW // _L, _L), jnp.int32),
        pltpu.VMEM((_HR * _L,), jnp.float32),
        pltpu.VMEM((_NS, _HSL), jnp.float32),
        pltpu.VMEM((_HSL, _L), jnp.float32),
        pltpu.VMEM_SHARED((_NS, _HR * _L), jnp.float32),
    ],
    compiler_params=pltpu.CompilerParams(
        needs_layout_passes=False, use_tc_tiling_on_sc=False),
)
def _sc_deg(edge16_hbm, out_hbm, idx_v, hist_v, tmp_v, rep_v, stage_sh):
    cid = lax.axis_index("c")
    sid = lax.axis_index("s")
    wid = sid * _NC + cid
    epw = _E // _NW
    zero16 = jnp.zeros((_L,), jnp.float32)

    def zbody(i, _):
        hist_v[pl.ds(i * _L, _L)] = zero16
        return 0
    lax.fori_loop(0, _HR, zbody, 0)

    pltpu.sync_copy(edge16_hbm.at[1, pl.ds(wid * (epw // _L), epw // _L)],
                    idx_v)
    ones16 = jnp.ones((_L,), jnp.float32)

    def body(i, _):
        ii = idx_v[i]
        plsc.addupdate_scatter(hist_v, [ii], ones16)
        return 0
    lax.fori_loop(0, epw // _L, body, 0)

    pltpu.sync_copy(hist_v, stage_sh.at[sid])
    plsc.subcore_barrier()
    pltpu.sync_copy(stage_sh.at[:, pl.ds(sid * _HSL, _HSL)], tmp_v)

    def rbody(k, _):
        acc = tmp_v[0, pl.ds(k * _L, _L)]
        for t in range(1, _NS):
            acc = acc + tmp_v[t, pl.ds(k * _L, _L)]
        tmp_v[0, pl.ds(k * _L, _L)] = acc
        return 0
    lax.fori_loop(0, _HSL // _L, rbody, 0)

    lane = lax.iota(jnp.int32, _L)

    def bbody(k, _):
        v = tmp_v[0, pl.ds(k * _L, _L)]
        row = k * _L + lane
        for j in range(_L):
            col = jnp.full((_L,), j, jnp.int32)
            plsc.store_scatter(rep_v, [row, col], v)
        return 0
    lax.fori_loop(0, _HSL // _L, bbody, 0)
    pltpu.sync_copy(rep_v, out_hbm.at[cid, pl.ds(sid * _HSL, _HSL)])


def _make_agg(F, nbuf):
    @functools.partial(
        pl.kernel,
        out_type=jax.ShapeDtypeStruct((_NC, _NROWS, F), jnp.float32),
        mesh=_mesh(),
        scratch_types=[
            pltpu.VMEM((_CMAX, _CHUNK), jnp.int32),
            pltpu.VMEM((_CMAX, _CHUNK), jnp.int32),
            pltpu.VMEM((nbuf, _CHUNK, F), jnp.float32),
            pltpu.VMEM_SHARED((_NROWS, F), jnp.float32),
            pltpu.SemaphoreType.DMA,
            pltpu.SemaphoreType.DMA,
        ],
        compiler_params=pltpu.CompilerParams(
            needs_layout_passes=False, use_tc_tiling_on_sc=False),
    )
    def agg(edge3_hbm, tab_hbm, out_hbm, srcv, dstv, rows, acc_sh,
            gsem, ssem):
        cid = lax.axis_index("c")
        sid = lax.axis_index("s")
        wid = sid * _NC + cid
        base, nsc = _worker_chunks(wid)
        zero16 = jnp.zeros((_L,), jnp.float32)

        def zbody(i, _):
            for k in range(F // _L):
                rows[0, i, pl.ds(k * _L, _L)] = zero16
            return 0
        lax.fori_loop(0, 128, zbody, 0)

        def zcopy(j, _):
            pltpu.sync_copy(
                rows.at[0, pl.ds(0, 128)],
                acc_sh.at[pl.ds(sid * _RPT + j * 128, 128)])
            return 0
        lax.fori_loop(0, _RPT // 128, zcopy, 0)

        pltpu.sync_copy(edge3_hbm.at[0, pl.ds(base, _CMAX)], srcv)
        pltpu.sync_copy(edge3_hbm.at[1, pl.ds(base, _CMAX)], dstv)
        plsc.subcore_barrier()

        def g_idx(c):
            return srcv.at[c]

        def s_idx(c):
            return dstv.at[c]

        for b in range(nbuf):
            pltpu.async_copy(tab_hbm.at[g_idx(b)], rows.at[b], gsem)

        def body(c, _):
            b = lax.rem(c, nbuf)
            pltpu.make_async_copy(tab_hbm.at[g_idx(c)], rows.at[b],
                                  gsem).wait()
            pltpu.async_copy(rows.at[b], acc_sh.at[s_idx(c)], ssem,
                             add=True)

            @pl.when(c >= 1)
            def _refill():
                p = c - 1
                pb = lax.rem(p, nbuf)
                pltpu.make_async_copy(rows.at[pb], acc_sh.at[s_idx(p)],
                                      ssem).wait()

                @pl.when(p + nbuf < nsc)
                def _issue():
                    pltpu.async_copy(tab_hbm.at[g_idx(p + nbuf)],
                                     rows.at[pb], gsem)
            return 0
        lax.fori_loop(0, nsc, body, 0)

        pltpu.make_async_copy(rows.at[lax.rem(nsc - 1, nbuf)],
                              acc_sh.at[s_idx(nsc - 1)], ssem).wait()
        plsc.subcore_barrier()
        pltpu.sync_copy(acc_sh.at[pl.ds(sid * _RPT, _RPT)],
                        out_hbm.at[cid, pl.ds(sid * _RPT, _RPT)])
    return agg


_sc_agg16 = _make_agg(_DIM, 8)
_sc_agg64 = _make_agg(_C, 6)


def _tc_mm1(x_ref, w_ref, o_ref):
    o_ref[...] = jnp.dot(x_ref[...], w_ref[...],
                         preferred_element_type=jnp.float32)


def _tc_prep(deg_ref, h_ref, hs_ref, dinv_ref):
    dsum = deg_ref[0, pl.ds(0, _N), :] + deg_ref[1, pl.ds(0, _N), :] + 1.0
    dinv = lax.rsqrt(dsum)
    dinv_ref[...] = dinv
    hs_ref[...] = h_ref[...] * dinv


def _tc_mid(a_ref, hs_ref, dinv_ref, w_ref, b_ref, gs_ref):
    dinv = dinv_ref[...]
    a = a_ref[0, pl.ds(0, _N), :] + a_ref[1, pl.ds(0, _N), :]
    pre = (a + hs_ref[...]) * dinv + b_ref[...]
    h1 = jnp.maximum(pre, 0.0)
    g = jnp.dot(h1, w_ref[...], preferred_element_type=jnp.float32)
    gs_ref[...] = g * dinv[:, 0:1]


def _tc_final(a_ref, gs_ref, dinv_ref, b_ref, logp_ref, out_ref):
    dinv = dinv_ref[:, 0:1]
    a = a_ref[0, pl.ds(0, _N), :] + a_ref[1, pl.ds(0, _N), :]
    out = (a + gs_ref[...]) * dinv + b_ref[...]
    m = jnp.max(out, axis=1, keepdims=True)
    lse = jnp.log(jnp.sum(jnp.exp(out - m), axis=1, keepdims=True)) + m
    out_ref[...] = out
    logp_ref[...] = out - lse


@jax.jit
def _run(x, edge_index, W1, b1, W2, b2):
    edge3 = edge_index.reshape(2, _TOTC, _CHUNK)
    edge16 = edge_index.reshape(2, _E // _L, _L)

    degp = _sc_deg(edge16)
    h = pl.pallas_call(
        _tc_mm1,
        out_shape=jax.ShapeDtypeStruct((_N, _DIM), jnp.float32),
    )(x, W1)

    hs, dinv = pl.pallas_call(
        _tc_prep,
        out_shape=(jax.ShapeDtypeStruct((_N, _DIM), jnp.float32),
                   jax.ShapeDtypeStruct((_N, _DIM), jnp.float32)),
    )(degp, h)

    a1 = _sc_agg16(edge3, hs)
    gs = pl.pallas_call(
        _tc_mid,
        out_shape=jax.ShapeDtypeStruct((_N, _C), jnp.float32),
    )(a1, hs, dinv, W2, b1.reshape(1, _DIM))

    a2 = _sc_agg64(edge3, gs)
    logp, out = pl.pallas_call(
        _tc_final,
        out_shape=(jax.ShapeDtypeStruct((_N, _C), jnp.float32),
                   jax.ShapeDtypeStruct((_N, _C), jnp.float32)),
    )(a2, gs, dinv, b2.reshape(1, _C))
    return logp, out


def kernel(x, edge_index, W1, b1, W2, b2):
    return _run(x, edge_index, W1, b1, W2, b2)


# speedup vs baseline: 1.2608x; 1.1811x over previous
import functools
import jax
import jax.numpy as jnp
from jax import lax
from jax.experimental import pallas as pl
from jax.experimental.pallas import tpu as pltpu
from jax.experimental.pallas import tpu_sc as plsc

_N = 10000
_E = 320000
_FIN, _DIM, _C = 128, 16, 64
_NC, _NS, _L = 2, 16, 16
_NW = _NC * _NS
_CHUNK = 128
_TOTC = _E // _CHUNK
_CPW = _TOTC // _NW
_XTRA = _TOTC - _CPW * _NW
_CMAX = _CPW + 1
_NROWS = 10240
_RPT = _NROWS // _NS
_HR = 640
_HSL = _HR * _L // _NS
_PN = _N // 8
_PR = _NROWS // 8


def _mesh():
    return plsc.VectorSubcoreMesh(
        core_axis_name="c", subcore_axis_name="s",
        num_cores=_NC, num_subcores=_NS)


def _worker_chunks(wid):
    first_x = _NW - _XTRA
    base = _CPW * wid + lax.max(wid - first_x, 0)
    n = jnp.where(wid >= first_x, _CMAX, _CPW)
    return base, n


@functools.partial(
    pl.kernel,
    out_type=(jax.ShapeDtypeStruct((_NC, _HR * _L, _L), jnp.float32),
              jax.ShapeDtypeStruct((_NC, _HR * _L, _C), jnp.float32)),
    mesh=_mesh(),
    scratch_types=[
        pltpu.VMEM((_E // _NW // _L, _L), jnp.int32),
        pltpu.VMEM((_HR * _L,), jnp.float32),
        pltpu.VMEM((_NS, _HSL), jnp.float32),
        pltpu.VMEM((_HSL, _L), jnp.float32),
        pltpu.VMEM_SHARED((_NS, _HR * _L), jnp.float32),
    ],
    compiler_params=pltpu.CompilerParams(
        needs_layout_passes=False, use_tc_tiling_on_sc=False),
)
def _sc_deg(edge16_hbm, out16_hbm, out64_hbm, idx_v, hist_v, tmp_v, rep_v,
            stage_sh):
    cid = lax.axis_index("c")
    sid = lax.axis_index("s")
    wid = sid * _NC + cid
    epw = _E // _NW
    zero16 = jnp.zeros((_L,), jnp.float32)

    def zbody(i, _):
        hist_v[pl.ds(i * _L, _L)] = zero16
        return 0
    lax.fori_loop(0, _HR, zbody, 0)

    pltpu.sync_copy(edge16_hbm.at[1, pl.ds(wid * (epw // _L), epw // _L)],
                    idx_v)
    ones16 = jnp.ones((_L,), jnp.float32)

    def body(i, _):
        ii = idx_v[i]
        plsc.addupdate_scatter(hist_v, [ii], ones16)
        return 0
    lax.fori_loop(0, epw // _L, body, 0)

    pltpu.sync_copy(hist_v, stage_sh.at[sid])
    plsc.subcore_barrier()
    pltpu.sync_copy(stage_sh.at[:, pl.ds(sid * _HSL, _HSL)], tmp_v)

    def rbody(k, _):
        acc = tmp_v[0, pl.ds(k * _L, _L)]
        for t in range(1, _NS):
            acc = acc + tmp_v[t, pl.ds(k * _L, _L)]
        tmp_v[0, pl.ds(k * _L, _L)] = acc
        return 0
    lax.fori_loop(0, _HSL // _L, rbody, 0)

    lane = lax.iota(jnp.int32, _L)

    def bbody(k, _):
        v = tmp_v[0, pl.ds(k * _L, _L)]
        row = k * _L + lane
        for j in range(_L):
            col = jnp.full((_L,), j, jnp.int32)
            plsc.store_scatter(rep_v, [row, col], v)
        return 0
    lax.fori_loop(0, _HSL // _L, bbody, 0)
    pltpu.sync_copy(rep_v, out16_hbm.at[cid, pl.ds(sid * _HSL, _HSL)])
    for j in range(_C // _L):
        pltpu.sync_copy(
            rep_v,
            out64_hbm.at[cid, pl.ds(sid * _HSL, _HSL), pl.ds(j * _L, _L)])


def _make_agg(F, nbuf):
    @functools.partial(
        pl.kernel,
        out_type=jax.ShapeDtypeStruct((_NC, _NROWS, F), jnp.float32),
        mesh=_mesh(),
        scratch_types=[
            pltpu.VMEM((_CMAX, _CHUNK), jnp.int32),
            pltpu.VMEM((_CMAX, _CHUNK), jnp.int32),
            pltpu.VMEM((nbuf, _CHUNK, F), jnp.float32),
            pltpu.VMEM_SHARED((_NROWS, F), jnp.float32),
            pltpu.SemaphoreType.DMA,
            pltpu.SemaphoreType.DMA,
        ],
        compiler_params=pltpu.CompilerParams(
            needs_layout_passes=False, use_tc_tiling_on_sc=False),
    )
    def agg(edge3_hbm, tab_hbm, out_hbm, srcv, dstv, rows, acc_sh,
            gsem, ssem):
        cid = lax.axis_index("c")
        sid = lax.axis_index("s")
        wid = sid * _NC + cid
        base, nsc = _worker_chunks(wid)
        zero16 = jnp.zeros((_L,), jnp.float32)

        def zbody(i, _):
            for k in range(F // _L):
                rows[0, i, pl.ds(k * _L, _L)] = zero16
            return 0
        lax.fori_loop(0, 128, zbody, 0)

        def zcopy(j, _):
            pltpu.sync_copy(
                rows.at[0, pl.ds(0, 128)],
                acc_sh.at[pl.ds(sid * _RPT + j * 128, 128)])
            return 0
        lax.fori_loop(0, _RPT // 128, zcopy, 0)

        pltpu.sync_copy(edge3_hbm.at[0, pl.ds(base, _CMAX)], srcv)
        pltpu.sync_copy(edge3_hbm.at[1, pl.ds(base, _CMAX)], dstv)
        plsc.subcore_barrier()

        def g_idx(c):
            return srcv.at[c]

        def s_idx(c):
            return dstv.at[c]

        for b in range(nbuf):
            pltpu.async_copy(tab_hbm.at[g_idx(b)], rows.at[b], gsem)

        def body(c, _):
            b = lax.rem(c, nbuf)
            pltpu.make_async_copy(tab_hbm.at[g_idx(c)], rows.at[b],
                                  gsem).wait()
            pltpu.async_copy(rows.at[b], acc_sh.at[s_idx(c)], ssem,
                             add=True)

            @pl.when(c >= 1)
            def _refill():
                p = c - 1
                pb = lax.rem(p, nbuf)
                pltpu.make_async_copy(rows.at[pb], acc_sh.at[s_idx(p)],
                                      ssem).wait()

                @pl.when(p + nbuf < nsc)
                def _issue():
                    pltpu.async_copy(tab_hbm.at[g_idx(p + nbuf)],
                                     rows.at[pb], gsem)
            return 0
        lax.fori_loop(0, nsc, body, 0)

        pltpu.make_async_copy(rows.at[lax.rem(nsc - 1, nbuf)],
                              acc_sh.at[s_idx(nsc - 1)], ssem).wait()
        plsc.subcore_barrier()
        pltpu.sync_copy(acc_sh.at[pl.ds(sid * _RPT, _RPT)],
                        out_hbm.at[cid, pl.ds(sid * _RPT, _RPT)])
    return agg


_sc_agg16 = _make_agg(_DIM, 8)
_sc_agg64 = _make_agg(_C, 6)


def _tc_mm1(x_ref, w_ref, o_ref):
    o_ref[...] = jnp.dot(x_ref[...], w_ref[...],
                         preferred_element_type=jnp.float32)


def _tc_prep(deg_ref, h_ref, hs_ref, dinv_ref):
    dsum = deg_ref[0, pl.ds(0, _PN), :] + deg_ref[1, pl.ds(0, _PN), :] + 1.0
    dinv = lax.rsqrt(dsum)
    dinv_ref[...] = dinv
    hs_ref[...] = h_ref[...] * dinv


def _tc_mid(a_ref, hs_ref, dinv_ref, w_ref, b_ref, gs_ref):
    dinv = dinv_ref[...]
    a = a_ref[0, pl.ds(0, _PN), :] + a_ref[1, pl.ds(0, _PN), :]
    pre = (a + hs_ref[...]) * dinv + b_ref[...]
    h1s = jnp.maximum(pre, 0.0) * dinv
    gs_ref[...] = jnp.dot(h1s, w_ref[...], preferred_element_type=jnp.float32)


def _tc_final(a_ref, gs_ref, deg_ref, b_ref, logp_ref, out_ref):
    dsum = deg_ref[0, pl.ds(0, _PN), :] + deg_ref[1, pl.ds(0, _PN), :] + 1.0
    dinv = lax.rsqrt(dsum)
    a = a_ref[0, pl.ds(0, _PN), :] + a_ref[1, pl.ds(0, _PN), :]
    out = (a + gs_ref[...]) * dinv + b_ref[...]
    for j in range(8):
        blk = out[:, j * _C:(j + 1) * _C]
        m = jnp.max(blk, axis=1, keepdims=True)
        lse = jnp.log(jnp.sum(jnp.exp(blk - m), axis=1, keepdims=True)) + m
        out_ref[:, j * _C:(j + 1) * _C] = blk
        logp_ref[:, j * _C:(j + 1) * _C] = blk - lse


@jax.jit
def _run(x, edge_index, W1, b1, W2, b2):
    edge3 = edge_index.reshape(2, _TOTC, _CHUNK)
    edge16 = edge_index.reshape(2, _E // _L, _L)
    eye8 = jnp.eye(8, dtype=jnp.float32)
    W1b = jnp.kron(eye8, W1)
    W2b = jnp.kron(eye8, W2)
    b1t = jnp.tile(b1, 8).reshape(1, 8 * _DIM)
    b2t = jnp.tile(b2, 8).reshape(1, 8 * _C)

    deg16, deg64 = _sc_deg(edge16)
    deg16_p = deg16.reshape(_NC, _PR, 8 * _DIM)
    deg64_p = deg64.reshape(_NC, _PR, 8 * _C)

    h_p = pl.pallas_call(
        _tc_mm1,
        out_shape=jax.ShapeDtypeStruct((_PN, 8 * _DIM), jnp.float32),
    )(x.reshape(_PN, 8 * _FIN), W1b)

    hs_p, dinv_p = pl.pallas_call(
        _tc_prep,
        out_shape=(jax.ShapeDtypeStruct((_PN, 8 * _DIM), jnp.float32),
                   jax.ShapeDtypeStruct((_PN, 8 * _DIM), jnp.float32)),
    )(deg16_p, h_p)

    a1 = _sc_agg16(edge3, hs_p.reshape(_N, _DIM))
    gs_p = pl.pallas_call(
        _tc_mid,
        out_shape=jax.ShapeDtypeStruct((_PN, 8 * _C), jnp.float32),
    )(a1.reshape(_NC, _PR, 8 * _DIM), hs_p, dinv_p, W2b, b1t)

    a2 = _sc_agg64(edge3, gs_p.reshape(_N, _C))
    logp_p, out_p = pl.pallas_call(
        _tc_final,
        out_shape=(jax.ShapeDtypeStruct((_PN, 8 * _C), jnp.float32),
                   jax.ShapeDtypeStruct((_PN, 8 * _C), jnp.float32)),
    )(a2.reshape(_NC, _PR, 8 * _C), gs_p, deg64_p, b2t)
    return logp_p.reshape(_N, _C), out_p.reshape(_N, _C)


def kernel(x, edge_index, W1, b1, W2, b2):
    return _run(x, edge_index, W1, b1, W2, b2)
